# Initial kernel scaffold; baseline (speedup 1.0000x reference)
import jax
import jax.numpy as jnp
from jax.experimental import pallas as pl


def kernel(x, emb):
    raise NotImplementedError("write your pallas kernel here")


# speedup vs baseline: 1.6677x; 1.6677x over previous
import jax
import jax.numpy as jnp
from jax.experimental import pallas as pl

_SEQ_BLK = 1024


def _add_kernel(x_ref, emb_ref, o_ref):
    o_ref[...] = x_ref[...] + emb_ref[...]


def kernel(x, emb):
    B, S, D = x.shape
    grid = (S // _SEQ_BLK, B)
    return pl.pallas_call(
        _add_kernel,
        grid=grid,
        in_specs=[
            pl.BlockSpec((1, _SEQ_BLK, D), lambda i, j: (j, i, 0)),
            pl.BlockSpec((_SEQ_BLK, D), lambda i, j: (i, 0)),
        ],
        out_specs=pl.BlockSpec((1, _SEQ_BLK, D), lambda i, j: (j, i, 0)),
        out_shape=jax.ShapeDtypeStruct((B, S, D), x.dtype),
    )(x, emb)


# speedup vs baseline: 1.7405x; 1.0437x over previous
import jax
import jax.numpy as jnp
from jax.experimental import pallas as pl

_SEQ_BLK = 2048


def _add_kernel(x_ref, emb_ref, o_ref):
    o_ref[...] = x_ref[...] + emb_ref[...]


def kernel(x, emb):
    B, S, D = x.shape
    grid = (S // _SEQ_BLK, B)
    return pl.pallas_call(
        _add_kernel,
        grid=grid,
        in_specs=[
            pl.BlockSpec((1, _SEQ_BLK, D), lambda i, j: (j, i, 0)),
            pl.BlockSpec((_SEQ_BLK, D), lambda i, j: (i, 0)),
        ],
        out_specs=pl.BlockSpec((1, _SEQ_BLK, D), lambda i, j: (j, i, 0)),
        out_shape=jax.ShapeDtypeStruct((B, S, D), x.dtype),
    )(x, emb)
